# baseline (speedup 1.0000x reference)
import functools

import jax
import jax.numpy as jnp
from jax import lax
from jax.experimental import pallas as pl
from jax.experimental.pallas import tpu as pltpu
from jax.experimental.pallas import tpu_sc as plsc

B0, B1, B2 = 256, 1024, 4096
VOCAB = 1000
VPAD = 1024
BATCH = 16384
DIVISOR = 10001

ROWS = 128
NBLK = B2 // ROWS


def _table_body(w_ref, w0_ref, w1_ref, w2_ref, t_ref):
    r4 = lax.broadcasted_iota(jnp.int32, (ROWS, ROWS // 4), 0) // 4
    c4 = lax.broadcasted_iota(jnp.int32, (ROWS, ROWS // 4), 1)
    s4 = (r4 == c4).astype(jnp.float32)
    r16 = lax.broadcasted_iota(jnp.int32, (ROWS, ROWS // 16), 0) // 16
    c16 = lax.broadcasted_iota(jnp.int32, (ROWS, ROWS // 16), 1)
    s16 = (r16 == c16).astype(jnp.float32)
    acc = w_ref[0] * jnp.dot(s16, w0_ref[...], preferred_element_type=jnp.float32)
    acc = acc + w_ref[1] * jnp.dot(s4, w1_ref[...], preferred_element_type=jnp.float32)
    acc = acc + w_ref[2] * w2_ref[...]
    acc = jnp.concatenate(
        [acc, jnp.zeros((ROWS, VPAD - VOCAB), jnp.float32)], axis=1)
    t_ref[...] = acc.reshape(ROWS // 8, 8, NT, 128)


def _build_table(w, W0, W1, W2):
    return pl.pallas_call(
        _table_body,
        grid=(NBLK,),
        in_specs=[
            pl.BlockSpec(memory_space=pltpu.SMEM),
            pl.BlockSpec((ROWS // 16, VOCAB), lambda i: (i, 0)),
            pl.BlockSpec((ROWS // 4, VOCAB), lambda i: (i, 0)),
            pl.BlockSpec((ROWS, VOCAB), lambda i: (i, 0)),
        ],
        out_specs=pl.BlockSpec((ROWS // 8, 8, NT, 128), lambda i: (i, 0, 0, 0)),
        out_shape=jax.ShapeDtypeStruct((B2 // 8, 8, NT, 128), jnp.float32),
    )(w, W0, W1, W2)


RG = 64
NRB = BATCH // (8 * RG)


def _relayout_body(in_ref, out_ref):
    out_ref[...] = (
        in_ref[...].transpose(0, 2, 1, 3).reshape(8 * RG, VPAD)[:, :VOCAB])


def _relayout(y4):
    return pl.pallas_call(
        _relayout_body,
        grid=(NRB,),
        in_specs=[pl.BlockSpec((RG, NT, 8, 128), lambda i: (i, 0, 0, 0))],
        out_specs=pl.BlockSpec((8 * RG, VOCAB), lambda i: (i, 0)),
        out_shape=jax.ShapeDtypeStruct((BATCH, VOCAB), jnp.float32),
    )(y4)


NC, NS, L = 2, 16, 16
NW = NC * NS
BPW = BATCH // NW
CH = 32
NCH = BPW // CH
GPC = CH // 8
NT = VPAD // 128


def _make_gather():
    mesh = plsc.VectorSubcoreMesh(core_axis_name="c", subcore_axis_name="s")

    @functools.partial(
        pl.kernel,
        mesh=mesh,
        compiler_params=pltpu.CompilerParams(use_tc_tiling_on_sc=False),
        out_type=jax.ShapeDtypeStruct((BATCH // 8, NT, 8, 128), jnp.float32),
        scratch_types=[
            pltpu.VMEM((BPW,), jnp.int32),
            pltpu.VMEM((CH, VPAD), jnp.float32),
            pltpu.VMEM((CH, VPAD), jnp.float32),
            pltpu.SemaphoreType.DMA,
            pltpu.SemaphoreType.DMA,
            pltpu.SemaphoreType.DMA,
        ],
    )
    def gather_k(x_hbm, t_hbm, out_hbm, idx_v, buf0, buf1, sem0, sem1, semw):
        wid = lax.axis_index("s") * NC + lax.axis_index("c")
        base = wid * BPW
        gbase = wid * (BPW // 8)
        pltpu.sync_copy(x_hbm.at[pl.ds(base, BPW)], idx_v)

        recip = jnp.float32(1.0 / DIVISOR)

        def bucketize(i, carry):
            off = pl.multiple_of(i * L, L)
            a = idx_v[pl.ds(off, L)] * B2
            q = (a.astype(jnp.float32) * recip).astype(jnp.int32)
            r = a - q * DIVISOR
            q = jnp.where(r >= DIVISOR, q + 1, q)
            q = jnp.where(r < 0, q - 1, q)
            idx_v[pl.ds(off, L)] = jnp.minimum(jnp.maximum(q, 0), B2 - 1)
            return carry

        lax.fori_loop(0, BPW // L, bucketize, 0)

        bufs = (buf0, buf1)
        sems = (sem0, sem1)

        def start(c):
            return pltpu.async_copy(
                t_hbm.at[idx_v.at[pl.ds(c * CH, CH)]], bufs[c % 2], sems[c % 2])

        def write_out(c):
            buf = bufs[c % 2]

            def wg(g, carry):
                dst_g = gbase + c * GPC + g
                pend = []
                for t in range(NT):
                    pend.append(pltpu.async_copy(
                        buf.at[pl.ds(8 * g, 8), pl.ds(128 * t, 128)],
                        out_hbm.at[dst_g, t], semw))
                for p in pend:
                    p.wait()
                return carry

            lax.fori_loop(0, GPC, wg, 0)

        cp = start(0)
        for c in range(NCH):
            nxt = start(c + 1) if c + 1 < NCH else None
            cp.wait()
            write_out(c)
            cp = nxt

    return gather_k


def kernel(x, W0, W1, W2, scale_weights):
    w = jax.nn.softmax(scale_weights, axis=0)
    T = _build_table(w, W0, W1, W2).reshape(B2, VPAD)
    y4 = _make_gather()(x, T)
    return _relayout(y4)


# speedup vs baseline: 1.3320x; 1.3320x over previous
import functools

import jax
import jax.numpy as jnp
from jax import lax
from jax.experimental import pallas as pl
from jax.experimental.pallas import tpu as pltpu
from jax.experimental.pallas import tpu_sc as plsc

B0, B1, B2 = 256, 1024, 4096
VOCAB = 1000
VPAD = 1024
BATCH = 16384
DIVISOR = 10001

ROWS = 128
NBLK = B2 // ROWS


def _table_body(w_ref, w0_ref, w1_ref, w2_ref, t_ref):
    r4 = lax.broadcasted_iota(jnp.int32, (ROWS, ROWS // 4), 0) // 4
    c4 = lax.broadcasted_iota(jnp.int32, (ROWS, ROWS // 4), 1)
    s4 = (r4 == c4).astype(jnp.float32)
    r16 = lax.broadcasted_iota(jnp.int32, (ROWS, ROWS // 16), 0) // 16
    c16 = lax.broadcasted_iota(jnp.int32, (ROWS, ROWS // 16), 1)
    s16 = (r16 == c16).astype(jnp.float32)
    acc = w_ref[0] * jnp.dot(s16, w0_ref[...], preferred_element_type=jnp.float32)
    acc = acc + w_ref[1] * jnp.dot(s4, w1_ref[...], preferred_element_type=jnp.float32)
    acc = acc + w_ref[2] * w2_ref[...]
    acc = jnp.concatenate(
        [acc, jnp.zeros((ROWS, VPAD - VOCAB), jnp.float32)], axis=1)
    t_ref[...] = acc.reshape(ROWS // 8, 8, NT, 128)


def _build_table(w, W0, W1, W2):
    return pl.pallas_call(
        _table_body,
        grid=(NBLK,),
        in_specs=[
            pl.BlockSpec(memory_space=pltpu.SMEM),
            pl.BlockSpec((ROWS // 16, VOCAB), lambda i: (i, 0)),
            pl.BlockSpec((ROWS // 4, VOCAB), lambda i: (i, 0)),
            pl.BlockSpec((ROWS, VOCAB), lambda i: (i, 0)),
        ],
        out_specs=pl.BlockSpec((ROWS // 8, 8, NT, 128), lambda i: (i, 0, 0, 0)),
        out_shape=jax.ShapeDtypeStruct((B2 // 8, 8, NT, 128), jnp.float32),
    )(w, W0, W1, W2)


RG = 64
NRB = BATCH // (8 * RG)


def _relayout_body(in_ref, out_ref):
    out_ref[...] = (
        in_ref[...].transpose(0, 2, 1, 3).reshape(8 * RG, VPAD)[:, :VOCAB])


def _relayout(y4):
    return pl.pallas_call(
        _relayout_body,
        grid=(NRB,),
        in_specs=[pl.BlockSpec((RG, NT, 8, 128), lambda i: (i, 0, 0, 0))],
        out_specs=pl.BlockSpec((8 * RG, VOCAB), lambda i: (i, 0)),
        out_shape=jax.ShapeDtypeStruct((BATCH, VOCAB), jnp.float32),
    )(y4)


NC, NS, L = 2, 16, 16
NW = NC * NS
BPW = BATCH // NW
CH = 32
NCH = BPW // CH
GPC = CH // 8
NT = VPAD // 128


def _make_gather():
    mesh = plsc.VectorSubcoreMesh(core_axis_name="c", subcore_axis_name="s")

    @functools.partial(
        pl.kernel,
        mesh=mesh,
        compiler_params=pltpu.CompilerParams(use_tc_tiling_on_sc=False),
        out_type=jax.ShapeDtypeStruct((BATCH // 8, NT, 8, 128), jnp.float32),
        scratch_types=[
            pltpu.VMEM((BPW,), jnp.int32),
            pltpu.VMEM((CH, VPAD), jnp.float32),
            pltpu.VMEM((CH, VPAD), jnp.float32),
            pltpu.VMEM((CH, VPAD), jnp.float32),
            pltpu.SemaphoreType.DMA,
            pltpu.SemaphoreType.DMA,
            pltpu.SemaphoreType.DMA,
            pltpu.SemaphoreType.DMA,
        ],
    )
    def gather_k(x_hbm, t_hbm, out_hbm, idx_v, buf0, buf1, buf2,
                 sem0, sem1, sem2, semw):
        wid = lax.axis_index("s") * NC + lax.axis_index("c")
        base = wid * BPW
        gbase = wid * (BPW // 8)
        pltpu.sync_copy(x_hbm.at[pl.ds(base, BPW)], idx_v)

        recip = jnp.float32(1.0 / DIVISOR)

        def bucketize(i, carry):
            off = pl.multiple_of(i * L, L)
            a = idx_v[pl.ds(off, L)] * B2
            q = (a.astype(jnp.float32) * recip).astype(jnp.int32)
            r = a - q * DIVISOR
            q = jnp.where(r >= DIVISOR, q + 1, q)
            q = jnp.where(r < 0, q - 1, q)
            idx_v[pl.ds(off, L)] = jnp.minimum(jnp.maximum(q, 0), B2 - 1)
            return carry

        lax.fori_loop(0, BPW // L, bucketize, 0)

        bufs = (buf0, buf1, buf2)
        sems = (sem0, sem1, sem2)

        def start(c):
            return pltpu.async_copy(
                t_hbm.at[idx_v.at[pl.ds(c * CH, CH)]], bufs[c % 3], sems[c % 3])

        def write_out(c):
            buf = bufs[c % 3]

            def wg(g, carry):
                dst_g = gbase + c * GPC + g
                pend = []
                for t in range(NT):
                    pend.append(pltpu.async_copy(
                        buf.at[pl.ds(8 * g, 8), pl.ds(128 * t, 128)],
                        out_hbm.at[dst_g, t], semw))
                for p in pend:
                    p.wait()
                return carry

            lax.fori_loop(0, GPC, wg, 0)

        pend2 = (start(0), start(1))
        for c in range(NCH):
            nxt = start(c + 2) if c + 2 < NCH else None
            pend2[0].wait()
            write_out(c)
            pend2 = (pend2[1], nxt)

    return gather_k


def kernel(x, W0, W1, W2, scale_weights):
    w = jax.nn.softmax(scale_weights, axis=0)
    T = _build_table(w, W0, W1, W2).reshape(B2, VPAD)
    y4 = _make_gather()(x, T)
    return y4.transpose(0, 2, 1, 3).reshape(BATCH, VPAD)[:, :VOCAB]


# speedup vs baseline: 1.3415x; 1.0072x over previous
import functools

import jax
import jax.numpy as jnp
from jax import lax
from jax.experimental import pallas as pl
from jax.experimental.pallas import tpu as pltpu
from jax.experimental.pallas import tpu_sc as plsc

B0, B1, B2 = 256, 1024, 4096
VOCAB = 1000
VPAD = 1024
BATCH = 16384
DIVISOR = 10001

ROWS = 128
NBLK = B2 // ROWS


def _table_body(w_ref, w0_ref, w1_ref, w2_ref, t_ref):
    r4 = lax.broadcasted_iota(jnp.int32, (ROWS, ROWS // 4), 0) // 4
    c4 = lax.broadcasted_iota(jnp.int32, (ROWS, ROWS // 4), 1)
    s4 = (r4 == c4).astype(jnp.float32)
    r16 = lax.broadcasted_iota(jnp.int32, (ROWS, ROWS // 16), 0) // 16
    c16 = lax.broadcasted_iota(jnp.int32, (ROWS, ROWS // 16), 1)
    s16 = (r16 == c16).astype(jnp.float32)
    acc = w_ref[0] * jnp.dot(s16, w0_ref[...], preferred_element_type=jnp.float32)
    acc = acc + w_ref[1] * jnp.dot(s4, w1_ref[...], preferred_element_type=jnp.float32)
    acc = acc + w_ref[2] * w2_ref[...]
    acc = jnp.concatenate(
        [acc, jnp.zeros((ROWS, VPAD - VOCAB), jnp.float32)], axis=1)
    t_ref[...] = acc.reshape(ROWS // 8, 8, NT, 128)


def _build_table(w, W0, W1, W2):
    return pl.pallas_call(
        _table_body,
        grid=(NBLK,),
        in_specs=[
            pl.BlockSpec(memory_space=pltpu.SMEM),
            pl.BlockSpec((ROWS // 16, VOCAB), lambda i: (i, 0)),
            pl.BlockSpec((ROWS // 4, VOCAB), lambda i: (i, 0)),
            pl.BlockSpec((ROWS, VOCAB), lambda i: (i, 0)),
        ],
        out_specs=pl.BlockSpec((ROWS // 8, 8, NT, 128), lambda i: (i, 0, 0, 0)),
        out_shape=jax.ShapeDtypeStruct((B2 // 8, 8, NT, 128), jnp.float32),
    )(w, W0, W1, W2)


RG = 64
NRB = BATCH // (8 * RG)


def _relayout_body(in_ref, out_ref):
    out_ref[...] = (
        in_ref[...].transpose(0, 2, 1, 3).reshape(8 * RG, VPAD)[:, :VOCAB])


def _relayout(y4):
    return pl.pallas_call(
        _relayout_body,
        grid=(NRB,),
        in_specs=[pl.BlockSpec((RG, NT, 8, 128), lambda i: (i, 0, 0, 0))],
        out_specs=pl.BlockSpec((8 * RG, VOCAB), lambda i: (i, 0)),
        out_shape=jax.ShapeDtypeStruct((BATCH, VOCAB), jnp.float32),
    )(y4)


NC, NS, L = 2, 16, 16
NW = NC * NS
BPW = BATCH // NW
CH = 16
NCH = BPW // CH
GPC = CH // 8
NT = VPAD // 128


def _make_gather():
    mesh = plsc.VectorSubcoreMesh(core_axis_name="c", subcore_axis_name="s")

    @functools.partial(
        pl.kernel,
        mesh=mesh,
        compiler_params=pltpu.CompilerParams(use_tc_tiling_on_sc=False),
        out_type=jax.ShapeDtypeStruct((BATCH // 8, NT, 8, 128), jnp.float32),
        scratch_types=[
            pltpu.VMEM((BPW,), jnp.int32),
            pltpu.VMEM((CH, VPAD), jnp.float32),
            pltpu.VMEM((CH, VPAD), jnp.float32),
            pltpu.VMEM((CH, VPAD), jnp.float32),
            pltpu.SemaphoreType.DMA,
            pltpu.SemaphoreType.DMA,
            pltpu.SemaphoreType.DMA,
            pltpu.SemaphoreType.DMA,
        ],
    )
    def gather_k(x_hbm, t_hbm, out_hbm, idx_v, buf0, buf1, buf2,
                 sem0, sem1, sem2, semw):
        wid = lax.axis_index("s") * NC + lax.axis_index("c")
        base = wid * BPW
        gbase = wid * (BPW // 8)
        pltpu.sync_copy(x_hbm.at[pl.ds(base, BPW)], idx_v)

        recip = jnp.float32(1.0 / DIVISOR)

        def bucketize(i, carry):
            off = pl.multiple_of(i * L, L)
            a = idx_v[pl.ds(off, L)] * B2
            q = (a.astype(jnp.float32) * recip).astype(jnp.int32)
            r = a - q * DIVISOR
            q = jnp.where(r >= DIVISOR, q + 1, q)
            q = jnp.where(r < 0, q - 1, q)
            idx_v[pl.ds(off, L)] = jnp.minimum(jnp.maximum(q, 0), B2 - 1)
            return carry

        lax.fori_loop(0, BPW // L, bucketize, 0)

        bufs = (buf0, buf1, buf2)
        sems = (sem0, sem1, sem2)

        def start(c):
            return pltpu.async_copy(
                t_hbm.at[idx_v.at[pl.ds(c * CH, CH)]], bufs[c % 3], sems[c % 3])

        def write_out(c):
            buf = bufs[c % 3]

            def wg(g, carry):
                dst_g = gbase + c * GPC + g
                pend = []
                for t in range(NT):
                    pend.append(pltpu.async_copy(
                        buf.at[pl.ds(8 * g, 8), pl.ds(128 * t, 128)],
                        out_hbm.at[dst_g, t], semw))
                for p in pend:
                    p.wait()
                return carry

            lax.fori_loop(0, GPC, wg, 0)

        pend2 = (start(0), start(1))
        for c in range(NCH):
            nxt = start(c + 2) if c + 2 < NCH else None
            pend2[0].wait()
            write_out(c)
            pend2 = (pend2[1], nxt)

    return gather_k


def kernel(x, W0, W1, W2, scale_weights):
    w = jax.nn.softmax(scale_weights, axis=0)
    T = _build_table(w, W0, W1, W2).reshape(B2, VPAD)
    y4 = _make_gather()(x, T)
    return y4.transpose(0, 2, 1, 3).reshape(BATCH, VPAD)[:, :VOCAB]


# speedup vs baseline: 1.3427x; 1.0009x over previous
import functools

import jax
import jax.numpy as jnp
from jax import lax
from jax.experimental import pallas as pl
from jax.experimental.pallas import tpu as pltpu
from jax.experimental.pallas import tpu_sc as plsc

B0, B1, B2 = 256, 1024, 4096
VOCAB = 1000
VPAD = 1024
BATCH = 16384
DIVISOR = 10001

ROWS = 128
NBLK = B2 // ROWS


def _table_body(w_ref, w0_ref, w1_ref, w2_ref, t_ref):
    r4 = lax.broadcasted_iota(jnp.int32, (ROWS, ROWS // 4), 0) // 4
    c4 = lax.broadcasted_iota(jnp.int32, (ROWS, ROWS // 4), 1)
    s4 = (r4 == c4).astype(jnp.float32)
    r16 = lax.broadcasted_iota(jnp.int32, (ROWS, ROWS // 16), 0) // 16
    c16 = lax.broadcasted_iota(jnp.int32, (ROWS, ROWS // 16), 1)
    s16 = (r16 == c16).astype(jnp.float32)
    acc = w_ref[0] * jnp.dot(s16, w0_ref[...], preferred_element_type=jnp.float32)
    acc = acc + w_ref[1] * jnp.dot(s4, w1_ref[...], preferred_element_type=jnp.float32)
    acc = acc + w_ref[2] * w2_ref[...]
    acc = jnp.concatenate(
        [acc, jnp.zeros((ROWS, VPAD - VOCAB), jnp.float32)], axis=1)
    t_ref[...] = acc.reshape(ROWS // 8, 8, NT, 128)


def _build_table(w, W0, W1, W2):
    return pl.pallas_call(
        _table_body,
        grid=(NBLK,),
        in_specs=[
            pl.BlockSpec(memory_space=pltpu.SMEM),
            pl.BlockSpec((ROWS // 16, VOCAB), lambda i: (i, 0)),
            pl.BlockSpec((ROWS // 4, VOCAB), lambda i: (i, 0)),
            pl.BlockSpec((ROWS, VOCAB), lambda i: (i, 0)),
        ],
        out_specs=pl.BlockSpec((ROWS // 8, 8, NT, 128), lambda i: (i, 0, 0, 0)),
        out_shape=jax.ShapeDtypeStruct((B2 // 8, 8, NT, 128), jnp.float32),
    )(w, W0, W1, W2)


RG = 64
NRB = BATCH // (8 * RG)


def _relayout_body(in_ref, out_ref):
    out_ref[...] = (
        in_ref[...].transpose(0, 2, 1, 3).reshape(8 * RG, VPAD)[:, :VOCAB])


def _relayout(y4):
    return pl.pallas_call(
        _relayout_body,
        grid=(NRB,),
        in_specs=[pl.BlockSpec((RG, NT, 8, 128), lambda i: (i, 0, 0, 0))],
        out_specs=pl.BlockSpec((8 * RG, VOCAB), lambda i: (i, 0)),
        out_shape=jax.ShapeDtypeStruct((BATCH, VOCAB), jnp.float32),
    )(y4)


NC, NS, L = 2, 16, 16
NW = NC * NS
BPW = BATCH // NW
CH = 16
NCH = BPW // CH
GPC = CH // 8
NT = VPAD // 128


def _make_gather():
    mesh = plsc.VectorSubcoreMesh(core_axis_name="c", subcore_axis_name="s")

    @functools.partial(
        pl.kernel,
        mesh=mesh,
        compiler_params=pltpu.CompilerParams(use_tc_tiling_on_sc=False),
        out_type=jax.ShapeDtypeStruct((BATCH // 8, NT, 8, 128), jnp.float32),
        scratch_types=[
            pltpu.VMEM((BPW,), jnp.int32),
            pltpu.VMEM((CH, VPAD), jnp.float32),
            pltpu.VMEM((CH, VPAD), jnp.float32),
            pltpu.VMEM((CH, VPAD), jnp.float32),
            pltpu.VMEM((CH, VPAD), jnp.float32),
            pltpu.SemaphoreType.DMA,
            pltpu.SemaphoreType.DMA,
            pltpu.SemaphoreType.DMA,
            pltpu.SemaphoreType.DMA,
            pltpu.SemaphoreType.DMA,
        ],
    )
    def gather_k(x_hbm, t_hbm, out_hbm, idx_v, buf0, buf1, buf2, buf3,
                 sem0, sem1, sem2, sem3, semw):
        wid = lax.axis_index("s") * NC + lax.axis_index("c")
        base = wid * BPW
        gbase = wid * (BPW // 8)
        pltpu.sync_copy(x_hbm.at[pl.ds(base, BPW)], idx_v)

        recip = jnp.float32(1.0 / DIVISOR)

        def bucketize(i, carry):
            off = pl.multiple_of(i * L, L)
            a = idx_v[pl.ds(off, L)] * B2
            q = (a.astype(jnp.float32) * recip).astype(jnp.int32)
            r = a - q * DIVISOR
            q = jnp.where(r >= DIVISOR, q + 1, q)
            q = jnp.where(r < 0, q - 1, q)
            idx_v[pl.ds(off, L)] = jnp.minimum(jnp.maximum(q, 0), B2 - 1)
            return carry

        lax.fori_loop(0, BPW // L, bucketize, 0)

        bufs = (buf0, buf1, buf2, buf3)
        sems = (sem0, sem1, sem2, sem3)

        def start(c):
            return pltpu.async_copy(
                t_hbm.at[idx_v.at[pl.ds(c * CH, CH)]], bufs[c % 4], sems[c % 4])

        def write_out(c):
            buf = bufs[c % 4]

            def wg(g, carry):
                dst_g = gbase + c * GPC + g
                pend = []
                for t in range(NT):
                    pend.append(pltpu.async_copy(
                        buf.at[pl.ds(8 * g, 8), pl.ds(128 * t, 128)],
                        out_hbm.at[dst_g, t], semw))
                for p in pend:
                    p.wait()
                return carry

            lax.fori_loop(0, GPC, wg, 0)

        pend3 = (start(0), start(1), start(2))
        for c in range(NCH):
            nxt = start(c + 3) if c + 3 < NCH else None
            pend3[0].wait()
            write_out(c)
            pend3 = (pend3[1], pend3[2], nxt)

    return gather_k


def kernel(x, W0, W1, W2, scale_weights):
    w = jax.nn.softmax(scale_weights, axis=0)
    T = _build_table(w, W0, W1, W2).reshape(B2, VPAD)
    y4 = _make_gather()(x, T)
    return y4.transpose(0, 2, 1, 3).reshape(BATCH, VPAD)[:, :VOCAB]


# speedup vs baseline: 1.3453x; 1.0019x over previous
import functools

import jax
import jax.numpy as jnp
from jax import lax
from jax.experimental import pallas as pl
from jax.experimental.pallas import tpu as pltpu
from jax.experimental.pallas import tpu_sc as plsc

B0, B1, B2 = 256, 1024, 4096
VOCAB = 1000
VPAD = 1024
BATCH = 16384
DIVISOR = 10001

ROWS = 128
NBLK = B2 // ROWS


def _table_body(w_ref, w0_ref, w1_ref, w2_ref, t_ref):
    r4 = lax.broadcasted_iota(jnp.int32, (ROWS, ROWS // 4), 0) // 4
    c4 = lax.broadcasted_iota(jnp.int32, (ROWS, ROWS // 4), 1)
    s4 = (r4 == c4).astype(jnp.float32)
    r16 = lax.broadcasted_iota(jnp.int32, (ROWS, ROWS // 16), 0) // 16
    c16 = lax.broadcasted_iota(jnp.int32, (ROWS, ROWS // 16), 1)
    s16 = (r16 == c16).astype(jnp.float32)
    acc = w_ref[0] * jnp.dot(s16, w0_ref[...], preferred_element_type=jnp.float32)
    acc = acc + w_ref[1] * jnp.dot(s4, w1_ref[...], preferred_element_type=jnp.float32)
    acc = acc + w_ref[2] * w2_ref[...]
    acc = jnp.concatenate(
        [acc, jnp.zeros((ROWS, VPAD - VOCAB), jnp.float32)], axis=1)
    t_ref[...] = acc.reshape(ROWS // 8, 8, NT, 128)


def _build_table(w, W0, W1, W2):
    return pl.pallas_call(
        _table_body,
        grid=(NBLK,),
        in_specs=[
            pl.BlockSpec(memory_space=pltpu.SMEM),
            pl.BlockSpec((ROWS // 16, VOCAB), lambda i: (i, 0)),
            pl.BlockSpec((ROWS // 4, VOCAB), lambda i: (i, 0)),
            pl.BlockSpec((ROWS, VOCAB), lambda i: (i, 0)),
        ],
        out_specs=pl.BlockSpec((ROWS // 8, 8, NT, 128), lambda i: (i, 0, 0, 0)),
        out_shape=jax.ShapeDtypeStruct((B2 // 8, 8, NT, 128), jnp.float32),
    )(w, W0, W1, W2)


NC, NS, L = 2, 16, 16
NW = NC * NS
BPW = BATCH // NW
CH = 16
NCH = BPW // CH
GPC = CH // 8
NT = VPAD // 128


def _make_gather():
    mesh = plsc.VectorSubcoreMesh(core_axis_name="c", subcore_axis_name="s")

    @functools.partial(
        pl.kernel,
        mesh=mesh,
        compiler_params=pltpu.CompilerParams(use_tc_tiling_on_sc=False),
        out_type=jax.ShapeDtypeStruct((BATCH // 8, NT, 8, 128), jnp.float32),
        scratch_types=[
            pltpu.VMEM((BPW,), jnp.int32),
            pltpu.VMEM((CH, VPAD), jnp.float32),
            pltpu.VMEM((CH, VPAD), jnp.float32),
            pltpu.VMEM((CH, VPAD), jnp.float32),
            pltpu.VMEM((CH, VPAD), jnp.float32),
            pltpu.SemaphoreType.DMA,
            pltpu.SemaphoreType.DMA,
            pltpu.SemaphoreType.DMA,
            pltpu.SemaphoreType.DMA,
            pltpu.SemaphoreType.DMA,
        ],
    )
    def gather_k(x_hbm, t_hbm, out_hbm, idx_v, buf0, buf1, buf2, buf3,
                 sem0, sem1, sem2, sem3, semw):
        wid = lax.axis_index("s") * NC + lax.axis_index("c")
        base = wid * BPW
        gbase = wid * (BPW // 8)
        pltpu.sync_copy(x_hbm.at[pl.ds(base, BPW)], idx_v)

        recip = jnp.float32(1.0 / DIVISOR)

        def bucketize(i, carry):
            off = pl.multiple_of(i * L, L)
            a = idx_v[pl.ds(off, L)] * B2
            q = (a.astype(jnp.float32) * recip).astype(jnp.int32)
            r = a - q * DIVISOR
            q = jnp.where(r >= DIVISOR, q + 1, q)
            q = jnp.where(r < 0, q - 1, q)
            idx_v[pl.ds(off, L)] = jnp.minimum(jnp.maximum(q, 0), B2 - 1)
            return carry

        lax.fori_loop(0, BPW // L, bucketize, 0)

        bufs = (buf0, buf1, buf2, buf3)
        sems = (sem0, sem1, sem2, sem3)

        def start(c):
            return pltpu.async_copy(
                t_hbm.at[idx_v.at[pl.ds(c * CH, CH)]], bufs[c % 4], sems[c % 4])

        def write_out(c):
            buf = bufs[c % 4]

            def wg(g, carry):
                dst_g = gbase + c * GPC + g
                pend = []
                for t in range(NT):
                    pend.append(pltpu.async_copy(
                        buf.at[pl.ds(8 * g, 8), pl.ds(128 * t, 128)],
                        out_hbm.at[dst_g, t], semw))
                for p in pend:
                    p.wait()
                return carry

            lax.fori_loop(0, GPC, wg, 0)

        pend3 = (start(0), start(1), start(2))
        for c in range(NCH):
            nxt = start(c + 3) if c + 3 < NCH else None
            pend3[0].wait()
            write_out(c)
            pend3 = (pend3[1], pend3[2], nxt)

    return gather_k


def kernel(x, W0, W1, W2, scale_weights):
    w = jax.nn.softmax(scale_weights, axis=0)
    T = _build_table(w, W0, W1, W2).reshape(B2, VPAD)
    y4 = _make_gather()(x, T)
    return y4.transpose(0, 2, 1, 3).reshape(BATCH, VPAD)[:, :VOCAB]
